# baseline (speedup 1.0000x reference)
import jax
import jax.numpy as jnp
from jax import lax
from jax.experimental import pallas as pl
from jax.experimental.pallas import tpu as pltpu
from jax.experimental.pallas import tpu_sc as plsc

_NC = 2
_NS = 16
_NW = _NC * _NS

_WROWS = 250000
_WDIM = 128
_RPW = 7800
_TAIL_BASE = _NW * _RPW
_TAIL_ROWS = _WROWS - _TAIL_BASE
_CHUNK = 120
_N_CHUNKS = _RPW // _CHUNK
_NBUF = 2


def _sc_copy(in_hbm, out_hbm, buf, load_sems, store_sems):
    wid = lax.axis_index("s") * _NC + lax.axis_index("c")
    base = pl.multiple_of(wid * _RPW, 8)

    store_copies = [None] * _N_CHUNKS
    for k in range(_N_CHUNKS):
        s = k % _NBUF
        if k >= _NBUF:
            store_copies[k - _NBUF].wait()
        src = in_hbm.at[pl.ds(base + k * _CHUNK, _CHUNK)]
        dst = out_hbm.at[pl.ds(base + k * _CHUNK, _CHUNK)]
        lc = pltpu.make_async_copy(src, buf.at[s], load_sems.at[s])
        lc.start()
        lc.wait()
        st = pltpu.make_async_copy(buf.at[s], dst, store_sems.at[s])
        st.start()
        store_copies[k] = st
    for k in range(_N_CHUNKS - _NBUF, _N_CHUNKS):
        store_copies[k].wait()

    @pl.when(wid == 0)
    def _():
        for i, (off, rows) in enumerate(
            ((0, 120), (120, 120), (240, 120), (360, 40))
        ):
            s = i % _NBUF
            tb = buf.at[s].at[pl.ds(0, rows)]
            pltpu.sync_copy(in_hbm.at[pl.ds(_TAIL_BASE + off, rows)], tb)
            pltpu.sync_copy(tb, out_hbm.at[pl.ds(_TAIL_BASE + off, rows)])


def kernel(embeddings):
    n_rows, dim = embeddings.shape
    wide = embeddings.reshape(_WROWS, _WDIM)
    mesh = plsc.VectorSubcoreMesh(core_axis_name="c", subcore_axis_name="s")
    run = pl.kernel(
        _sc_copy,
        out_type=jax.ShapeDtypeStruct(wide.shape, wide.dtype),
        mesh=mesh,
        scratch_types=[
            pltpu.VMEM((_NBUF, _CHUNK, _WDIM), jnp.float32),
            pltpu.SemaphoreType.DMA((_NBUF,)),
            pltpu.SemaphoreType.DMA((_NBUF,)),
        ],
    )
    return run(wide).reshape(n_rows, dim)


# speedup vs baseline: 1.0049x; 1.0049x over previous
import jax
import jax.numpy as jnp
from jax import lax
from jax.experimental import pallas as pl
from jax.experimental.pallas import tpu as pltpu
from jax.experimental.pallas import tpu_sc as plsc

_NC = 2
_NS = 16
_NW = _NC * _NS

_ROWS = 1000000
_DIM = 32
_RPW = (_ROWS // _NW) // 8 * 8
_TAIL_BASE = _NW * _RPW
_TAIL_ROWS = _ROWS - _TAIL_BASE
_CHUNK = 504
_N_CHUNKS = _RPW // _CHUNK
_NBUF = 2


def _sc_copy(in_hbm, out_hbm, buf, load_sems, store_sems):
    wid = lax.axis_index("s") * _NC + lax.axis_index("c")
    base = pl.multiple_of(wid * _RPW, 8)

    store_copies = [None] * _N_CHUNKS
    for k in range(_N_CHUNKS):
        s = k % _NBUF
        if k >= _NBUF:
            store_copies[k - _NBUF].wait()
        src = in_hbm.at[pl.ds(base + k * _CHUNK, _CHUNK)]
        dst = out_hbm.at[pl.ds(base + k * _CHUNK, _CHUNK)]
        lc = pltpu.make_async_copy(src, buf.at[s], load_sems.at[s])
        lc.start()
        lc.wait()
        st = pltpu.make_async_copy(buf.at[s], dst, store_sems.at[s])
        st.start()
        store_copies[k] = st
    for k in range(_N_CHUNKS - _NBUF, _N_CHUNKS):
        store_copies[k].wait()

    @pl.when(wid == 0)
    def _():
        tail = buf.at[0].at[pl.ds(0, _TAIL_ROWS)]
        pltpu.sync_copy(in_hbm.at[pl.ds(_TAIL_BASE, _TAIL_ROWS)], tail)
        pltpu.sync_copy(tail, out_hbm.at[pl.ds(_TAIL_BASE, _TAIL_ROWS)])


def kernel(embeddings):
    mesh = plsc.VectorSubcoreMesh(core_axis_name="c", subcore_axis_name="s")
    run = pl.kernel(
        _sc_copy,
        out_type=jax.ShapeDtypeStruct(embeddings.shape, embeddings.dtype),
        mesh=mesh,
        scratch_types=[
            pltpu.VMEM((_NBUF, _CHUNK, _DIM), jnp.float32),
            pltpu.SemaphoreType.DMA((_NBUF,)),
            pltpu.SemaphoreType.DMA((_NBUF,)),
        ],
        compiler_params=pltpu.CompilerParams(use_tc_tiling_on_sc=False),
    )
    return run(embeddings)


# speedup vs baseline: 1.1916x; 1.1858x over previous
import jax
import jax.numpy as jnp
from jax import lax
from jax.experimental import pallas as pl
from jax.experimental.pallas import tpu as pltpu
from jax.experimental.pallas import tpu_sc as plsc

_NC = 2
_NS = 16
_NW = _NC * _NS

_ROWS = 1000000
_DIM = 32
_RPW = (_ROWS // _NW) // 32 * 32
_TAIL_BASE = _NW * _RPW
_TAIL_ROWS = _ROWS - _TAIL_BASE
_CHUNK = 256
_N_CHUNKS = _RPW // _CHUNK
_NBUF = 2
_TCHUNK = 192


def _sc_copy(in_hbm, out_hbm, buf, load_sems, store_sems):
    wid = lax.axis_index("s") * _NC + lax.axis_index("c")
    base = pl.multiple_of(wid * _RPW, 32)

    def body(k, carry):
        s = lax.rem(k, _NBUF)
        off = base + k * _CHUNK

        @pl.when(k >= _NBUF)
        def _():
            pltpu.make_async_copy(
                buf.at[s],
                out_hbm.at[pl.ds(0, _CHUNK)],
                store_sems.at[s],
            ).wait()

        lc = pltpu.make_async_copy(
            in_hbm.at[pl.ds(off, _CHUNK)], buf.at[s], load_sems.at[s]
        )
        lc.start()
        lc.wait()
        pltpu.make_async_copy(
            buf.at[s], out_hbm.at[pl.ds(off, _CHUNK)], store_sems.at[s]
        ).start()
        return carry

    lax.fori_loop(0, _N_CHUNKS, body, 0)
    for s in range(_NBUF):
        pltpu.make_async_copy(
            buf.at[s], out_hbm.at[pl.ds(0, _CHUNK)], store_sems.at[s]
        ).wait()

    @pl.when(wid == 0)
    def _():
        for i in range(_TAIL_ROWS // _TCHUNK):
            tb = buf.at[0].at[pl.ds(0, _TCHUNK)]
            off = _TAIL_BASE + i * _TCHUNK
            pltpu.sync_copy(in_hbm.at[pl.ds(off, _TCHUNK)], tb)
            pltpu.sync_copy(tb, out_hbm.at[pl.ds(off, _TCHUNK)])


def kernel(embeddings):
    mesh = plsc.VectorSubcoreMesh(core_axis_name="c", subcore_axis_name="s")
    run = pl.kernel(
        _sc_copy,
        out_type=jax.ShapeDtypeStruct(embeddings.shape, embeddings.dtype),
        mesh=mesh,
        scratch_types=[
            pltpu.VMEM((_NBUF, _CHUNK, _DIM), jnp.float32),
            pltpu.SemaphoreType.DMA((_NBUF,)),
            pltpu.SemaphoreType.DMA((_NBUF,)),
        ],
    )
    return run(embeddings)
